# baseline (speedup 1.0000x reference)
import jax
import jax.numpy as jnp
from jax.experimental import pallas as pl
from jax.experimental.pallas import tpu as pltpu

BF = jnp.bfloat16
F32 = jnp.float32


def _gate_coeffs(h_bf, z_bf, wh_ref, wz_ref, b_ref):
    l = (
        jnp.dot(h_bf, wh_ref[...], preferred_element_type=F32)
        + jnp.dot(z_bf, wz_ref[...], preferred_element_type=F32)
        + b_ref[0:1, :]
    )
    l = jnp.where(l >= 0, l, 0.01 * l)
    l0 = l[:, 0:1]
    l1 = l[:, 1:2]
    mx = jnp.maximum(l0, l1)
    e0 = jnp.exp(l0 - mx)
    e1 = jnp.exp(l1 - mx)
    s = e0 + e1
    m0 = e0 / s
    m1 = e1 / s
    inv = 1.0 / jnp.maximum(jnp.sqrt(m0 * m0 + m1 * m1), 1e-12)
    return m0 * inv, m1 * inv


def _pass1(adj_f32, x_bf, w_bf, h_f32, wh, wz, bp, bm=400):
    m, k = adj_f32.shape
    n = w_bf.shape[1]
    kf = x_bf.shape[1]
    nsteps = m // bm

    def body(a_ref, x_ref, w_ref, h_ref, wh_ref, wz_ref, b_ref,
             z_ref, g_ref, abf_ref, t_ref):
        zf = jnp.maximum(
            jnp.dot(t_ref[...].astype(BF), w_ref[...],
                    preferred_element_type=F32), 0.0)
        z_bf = zf.astype(BF)
        z_ref[...] = z_bf
        hf = h_ref[...]
        m0, m1 = _gate_coeffs(hf.astype(BF), z_bf, wh_ref, wz_ref, b_ref)
        g_ref[...] = (m0 * zf + m1 * hf).astype(BF)

        a_bf = a_ref[...].astype(BF)
        abf_ref[...] = a_bf
        t_ref[...] = jnp.dot(a_bf, x_ref[...], preferred_element_type=F32)

    cur = lambda i: (jnp.minimum(i, nsteps - 1), 0)
    prev = lambda i: (jnp.maximum(i - 1, 0), 0)

    return pl.pallas_call(
        body,
        grid=(nsteps + 1,),
        in_specs=[
            pl.BlockSpec((bm, k), cur),
            pl.BlockSpec(x_bf.shape, lambda i: (0, 0)),
            pl.BlockSpec(w_bf.shape, lambda i: (0, 0)),
            pl.BlockSpec((bm, n), prev),
            pl.BlockSpec(wh.shape, lambda i: (0, 0)),
            pl.BlockSpec(wz.shape, lambda i: (0, 0)),
            pl.BlockSpec(bp.shape, lambda i: (0, 0)),
        ],
        out_specs=[
            pl.BlockSpec((bm, n), prev),
            pl.BlockSpec((bm, n), prev),
            pl.BlockSpec((bm, k), cur),
        ],
        out_shape=[
            jax.ShapeDtypeStruct((m, n), BF),
            jax.ShapeDtypeStruct((m, n), BF),
            jax.ShapeDtypeStruct((m, k), BF),
        ],
        scratch_shapes=[pltpu.VMEM((bm, kf), F32)],
    )(adj_f32, x_bf, w_bf, h_f32, wh, wz, bp)


def _pass_mid(adj_bf, f_bf, w_bf, h_f32, wh, wz, bp, w4_bf=None, bm=400):
    m, k = adj_bf.shape
    kf = f_bf.shape[1]
    n = w_bf.shape[1]
    nsteps = m // bm
    emit_p = w4_bf is not None

    def body(a_ref, f_ref, w_ref, h_ref, wh_ref, wz_ref, b_ref, *rest):
        if emit_p:
            w4_ref, z_ref, o_ref, t_ref = rest
        else:
            z_ref, o_ref, t_ref = rest
        zf = jnp.maximum(
            jnp.dot(t_ref[...].astype(BF), w_ref[...],
                    preferred_element_type=F32), 0.0)
        z_bf = zf.astype(BF)
        z_ref[...] = z_bf
        hf = h_ref[...]
        h_bf = hf.astype(BF)
        m0, m1 = _gate_coeffs(h_bf, z_bf, wh_ref, wz_ref, b_ref)
        if emit_p:
            zw = jnp.dot(z_bf, w4_ref[...],
                         preferred_element_type=F32)
            hw = jnp.dot(h_bf, w4_ref[...],
                         preferred_element_type=F32)
            o_ref[...] = (m0 * zw + m1 * hw).astype(BF)
        else:
            o_ref[...] = (m0 * zf + m1 * hf).astype(BF)
        t_ref[...] = jnp.dot(a_ref[...], f_ref[...],
                             preferred_element_type=F32)

    cur = lambda i: (jnp.minimum(i, nsteps - 1), 0)
    prev = lambda i: (jnp.maximum(i - 1, 0), 0)

    in_specs = [
        pl.BlockSpec((bm, k), cur),
        pl.BlockSpec(f_bf.shape, lambda i: (0, 0)),
        pl.BlockSpec(w_bf.shape, lambda i: (0, 0)),
        pl.BlockSpec((bm, n), prev),
        pl.BlockSpec(wh.shape, lambda i: (0, 0)),
        pl.BlockSpec(wz.shape, lambda i: (0, 0)),
        pl.BlockSpec(bp.shape, lambda i: (0, 0)),
    ]
    out_specs = [pl.BlockSpec((bm, n), prev)]
    out_shape = [jax.ShapeDtypeStruct((m, n), BF)]
    args = [adj_bf, f_bf, w_bf, h_f32, wh, wz, bp]
    if emit_p:
        in_specs.append(pl.BlockSpec(w4_bf.shape, lambda i: (0, 0)))
        args.append(w4_bf)
        nq = w4_bf.shape[1]
        out_specs.append(pl.BlockSpec((bm, nq), prev))
        out_shape.append(jax.ShapeDtypeStruct((m, nq), BF))
    else:
        out_specs.append(pl.BlockSpec((bm, n), prev))
        out_shape.append(jax.ShapeDtypeStruct((m, n), BF))

    return pl.pallas_call(
        body,
        grid=(nsteps + 1,),
        in_specs=in_specs,
        out_specs=out_specs,
        out_shape=out_shape,
        scratch_shapes=[pltpu.VMEM((bm, kf), F32)],
    )(*args)


def _pass4(adj_bf, p4_bf, z1, z2, z3, zb, wls, bl_pad, w5s, bm=400):
    m, k = adj_bf.shape
    n_z = [z1.shape[1], z2.shape[1], z3.shape[1], zb.shape[1]]
    nq = w5s[0].shape[1]
    nsteps = m // bm

    def body(a_ref, p_ref, z1_ref, z2_ref, z3_ref, zb_ref,
             wl1_ref, wl2_ref, wl3_ref, wl4_ref, wlz_ref, bl_ref,
             w51_ref, w52_ref, w53_ref, w54_ref, w5z_ref, q_ref, t_ref):
        z4f = jnp.maximum(t_ref[...], 0.0)
        z4_bf = z4f.astype(BF)
        z_blks = (z1_ref[...], z2_ref[...], z3_ref[...], z4_bf, zb_ref[...])
        wl_refs = (wl1_ref, wl2_ref, wl3_ref, wl4_ref, wlz_ref)
        w5_refs = (w51_ref, w52_ref, w53_ref, w54_ref, w5z_ref)

        t = bl_ref[0:1, :]
        for zc, wl_ref in zip(z_blks, wl_refs):
            t = t + jnp.dot(zc, wl_ref[...], preferred_element_type=F32)
        t = jnp.where(t >= 0, t, 0.01 * t)

        ls = [t[:, c:c + 1] for c in range(5)]
        mx = ls[0]
        for c in range(1, 5):
            mx = jnp.maximum(mx, ls[c])
        es = [jnp.exp(lc - mx) for lc in ls]
        s = es[0] + es[1] + es[2] + es[3] + es[4]
        us = [ec / s for ec in es]
        nrm = jnp.sqrt(us[0] ** 2 + us[1] ** 2 + us[2] ** 2
                       + us[3] ** 2 + us[4] ** 2)
        inv = 1.0 / jnp.maximum(nrm, 1e-12)

        acc = jnp.zeros((z4f.shape[0], nq), F32)
        for uc, zc, w5_ref in zip(us, z_blks, w5_refs):
            acc = acc + (uc * inv) * jnp.dot(
                zc, w5_ref[...], preferred_element_type=F32)
        q_ref[...] = acc.astype(BF)

        t_ref[...] = jnp.dot(a_ref[...], p_ref[...],
                             preferred_element_type=F32)

    cur = lambda i: (jnp.minimum(i, nsteps - 1), 0)
    prev = lambda i: (jnp.maximum(i - 1, 0), 0)

    in_specs = [
        pl.BlockSpec((bm, k), cur),
        pl.BlockSpec(p4_bf.shape, lambda i: (0, 0)),
        pl.BlockSpec((bm, n_z[0]), prev),
        pl.BlockSpec((bm, n_z[1]), prev),
        pl.BlockSpec((bm, n_z[2]), prev),
        pl.BlockSpec((bm, n_z[3]), prev),
    ]
    in_specs += [pl.BlockSpec(w.shape, lambda i: (0, 0)) for w in wls]
    in_specs += [pl.BlockSpec(bl_pad.shape, lambda i: (0, 0))]
    in_specs += [pl.BlockSpec(w.shape, lambda i: (0, 0)) for w in w5s]

    return pl.pallas_call(
        body,
        grid=(nsteps + 1,),
        in_specs=in_specs,
        out_specs=pl.BlockSpec((bm, nq), prev),
        out_shape=jax.ShapeDtypeStruct((m, nq), BF),
        scratch_shapes=[pltpu.VMEM((bm, nq), F32)],
    )(adj_bf, p4_bf, z1, z2, z3, zb, *wls, bl_pad, *w5s)


def _pass5(adj_bf, q_bf, bm=400):
    m, k = adj_bf.shape
    n = q_bf.shape[1]

    def body(a_ref, q_ref, o_ref):
        acc = jnp.dot(a_ref[...], q_ref[...], preferred_element_type=F32)
        mx = jnp.max(acc, axis=1, keepdims=True)
        e = jnp.exp(acc - mx)
        o_ref[...] = e / jnp.sum(e, axis=1, keepdims=True)

    return pl.pallas_call(
        body,
        grid=(m // bm,),
        in_specs=[
            pl.BlockSpec((bm, k), lambda i: (i, 0)),
            pl.BlockSpec(q_bf.shape, lambda i: (0, 0)),
        ],
        out_specs=pl.BlockSpec((bm, n), lambda i: (i, 0)),
        out_shape=jax.ShapeDtypeStruct((m, n), F32),
    )(adj_bf, q_bf)


def _pad_cols(w, n=128):
    return jnp.pad(w, ((0, 0), (0, n - w.shape[1]))).astype(BF)


def _bias_pad(b):
    return jnp.zeros((8, 128), F32).at[0, : b.shape[0]].set(b)


def kernel(x, adj, h1, h2, h3, z, W1, W2, W3, W4, W5,
           w1, b1, w2, b2, w3, b3, wl, bl):
    ne1 = W1.shape[1]
    ne2 = W2.shape[1]
    ne3 = W3.shape[1]
    nz = W4.shape[1]

    z1, g1, adj_bf = _pass1(
        adj, x.astype(BF), W1.astype(BF), h1,
        _pad_cols(w1[:ne1]), _pad_cols(w1[ne1:]), _bias_pad(b1))

    z2, g2 = _pass_mid(
        adj_bf, g1, W2.astype(BF), h2,
        _pad_cols(w2[:ne2]), _pad_cols(w2[ne2:]), _bias_pad(b2))

    z3, p4 = _pass_mid(
        adj_bf, g2, W3.astype(BF), h3,
        _pad_cols(w3[:ne3]), _pad_cols(w3[ne3:]), _bias_pad(b3),
        w4_bf=W4.astype(BF), bm=200)

    splits = [ne1, ne1 + ne2, ne1 + ne2 + ne3, ne1 + ne2 + ne3 + nz]
    wls = [_pad_cols(p).astype(BF) for p in jnp.split(wl, splits, axis=0)]
    w5s = [p.astype(BF) for p in jnp.split(W5, splits, axis=0)]
    q = _pass4(adj_bf, p4, z1, z2, z3, z.astype(BF), wls, _bias_pad(bl), w5s)

    return _pass5(adj_bf, q)


# speedup vs baseline: 1.0023x; 1.0023x over previous
import jax
import jax.numpy as jnp
from jax.experimental import pallas as pl
from jax.experimental.pallas import tpu as pltpu

BF = jnp.bfloat16
F32 = jnp.float32


def _gate_coeffs(h_bf, z_bf, wh_ref, wz_ref, b_ref):
    l = (
        jnp.dot(h_bf, wh_ref[...], preferred_element_type=F32)
        + jnp.dot(z_bf, wz_ref[...], preferred_element_type=F32)
        + b_ref[0:1, :]
    )
    l = jnp.where(l >= 0, l, 0.01 * l)
    l0 = l[:, 0:1]
    l1 = l[:, 1:2]
    mx = jnp.maximum(l0, l1)
    e0 = jnp.exp(l0 - mx)
    e1 = jnp.exp(l1 - mx)
    s = e0 + e1
    m0 = e0 / s
    m1 = e1 / s
    inv = 1.0 / jnp.maximum(jnp.sqrt(m0 * m0 + m1 * m1), 1e-12)
    return m0 * inv, m1 * inv


def _pass1(adj_f32, x_bf, w_bf, h_f32, wh, wz, bp, wl_bf, w5_bf, bm=400):
    m, k = adj_f32.shape
    n = w_bf.shape[1]
    kf = x_bf.shape[1]
    nq = w5_bf.shape[1]
    nsteps = m // bm

    def body(a_ref, x_ref, w_ref, h_ref, wh_ref, wz_ref, b_ref,
             wl_ref, w5_ref, g_ref, c_ref, d_ref, abf_ref, t_ref):
        zf = jnp.maximum(
            jnp.dot(t_ref[...].astype(BF), w_ref[...],
                    preferred_element_type=F32), 0.0)
        z_bf = zf.astype(BF)
        c_ref[...] = jnp.dot(z_bf, wl_ref[...], preferred_element_type=F32)
        d_ref[...] = jnp.dot(z_bf, w5_ref[...], preferred_element_type=F32)
        hf = h_ref[...]
        m0, m1 = _gate_coeffs(hf.astype(BF), z_bf, wh_ref, wz_ref, b_ref)
        g_ref[...] = (m0 * zf + m1 * hf).astype(BF)

        a_bf = a_ref[...].astype(BF)
        abf_ref[...] = a_bf
        t_ref[...] = jnp.dot(a_bf, x_ref[...], preferred_element_type=F32)

    cur = lambda i: (jnp.minimum(i, nsteps - 1), 0)
    prev = lambda i: (jnp.maximum(i - 1, 0), 0)

    return pl.pallas_call(
        body,
        grid=(nsteps + 1,),
        in_specs=[
            pl.BlockSpec((bm, k), cur),
            pl.BlockSpec(x_bf.shape, lambda i: (0, 0)),
            pl.BlockSpec(w_bf.shape, lambda i: (0, 0)),
            pl.BlockSpec((bm, n), prev),
            pl.BlockSpec(wh.shape, lambda i: (0, 0)),
            pl.BlockSpec(wz.shape, lambda i: (0, 0)),
            pl.BlockSpec(bp.shape, lambda i: (0, 0)),
            pl.BlockSpec(wl_bf.shape, lambda i: (0, 0)),
            pl.BlockSpec(w5_bf.shape, lambda i: (0, 0)),
        ],
        out_specs=[
            pl.BlockSpec((bm, n), prev),
            pl.BlockSpec((bm, 128), prev),
            pl.BlockSpec((bm, nq), prev),
            pl.BlockSpec((bm, k), cur),
        ],
        out_shape=[
            jax.ShapeDtypeStruct((m, n), BF),
            jax.ShapeDtypeStruct((m, 128), F32),
            jax.ShapeDtypeStruct((m, nq), F32),
            jax.ShapeDtypeStruct((m, k), BF),
        ],
        scratch_shapes=[pltpu.VMEM((bm, kf), F32)],
    )(adj_f32, x_bf, w_bf, h_f32, wh, wz, bp, wl_bf, w5_bf)


def _pass_mid(adj_bf, f_bf, w_bf, h_f32, wh, wz, bp, wl_bf, w5_bf,
              w4_bf=None, bm=400):
    m, k = adj_bf.shape
    kf = f_bf.shape[1]
    n = w_bf.shape[1]
    nq = w5_bf.shape[1]
    nsteps = m // bm
    emit_p = w4_bf is not None

    def body(a_ref, f_ref, w_ref, h_ref, wh_ref, wz_ref, b_ref,
             wl_ref, w5_ref, *rest):
        if emit_p:
            w4_ref, o_ref, c_ref, d_ref, t_ref = rest
        else:
            o_ref, c_ref, d_ref, t_ref = rest
        zf = jnp.maximum(
            jnp.dot(t_ref[...].astype(BF), w_ref[...],
                    preferred_element_type=F32), 0.0)
        z_bf = zf.astype(BF)
        c_ref[...] = jnp.dot(z_bf, wl_ref[...], preferred_element_type=F32)
        d_ref[...] = jnp.dot(z_bf, w5_ref[...], preferred_element_type=F32)
        hf = h_ref[...]
        h_bf = hf.astype(BF)
        m0, m1 = _gate_coeffs(h_bf, z_bf, wh_ref, wz_ref, b_ref)
        if emit_p:
            zw = jnp.dot(z_bf, w4_ref[...],
                         preferred_element_type=F32)
            hw = jnp.dot(h_bf, w4_ref[...],
                         preferred_element_type=F32)
            o_ref[...] = (m0 * zw + m1 * hw).astype(BF)
        else:
            o_ref[...] = (m0 * zf + m1 * hf).astype(BF)
        t_ref[...] = jnp.dot(a_ref[...], f_ref[...],
                             preferred_element_type=F32)

    cur = lambda i: (jnp.minimum(i, nsteps - 1), 0)
    prev = lambda i: (jnp.maximum(i - 1, 0), 0)

    in_specs = [
        pl.BlockSpec((bm, k), cur),
        pl.BlockSpec(f_bf.shape, lambda i: (0, 0)),
        pl.BlockSpec(w_bf.shape, lambda i: (0, 0)),
        pl.BlockSpec((bm, n), prev),
        pl.BlockSpec(wh.shape, lambda i: (0, 0)),
        pl.BlockSpec(wz.shape, lambda i: (0, 0)),
        pl.BlockSpec(bp.shape, lambda i: (0, 0)),
        pl.BlockSpec(wl_bf.shape, lambda i: (0, 0)),
        pl.BlockSpec(w5_bf.shape, lambda i: (0, 0)),
    ]
    args = [adj_bf, f_bf, w_bf, h_f32, wh, wz, bp, wl_bf, w5_bf]
    if emit_p:
        in_specs.append(pl.BlockSpec(w4_bf.shape, lambda i: (0, 0)))
        args.append(w4_bf)
        no = w4_bf.shape[1]
        o_dtype = BF
    else:
        no = n
        o_dtype = BF
    out_specs = [
        pl.BlockSpec((bm, no), prev),
        pl.BlockSpec((bm, 128), prev),
        pl.BlockSpec((bm, nq), prev),
    ]
    out_shape = [
        jax.ShapeDtypeStruct((m, no), o_dtype),
        jax.ShapeDtypeStruct((m, 128), F32),
        jax.ShapeDtypeStruct((m, nq), F32),
    ]

    return pl.pallas_call(
        body,
        grid=(nsteps + 1,),
        in_specs=in_specs,
        out_specs=out_specs,
        out_shape=out_shape,
        scratch_shapes=[pltpu.VMEM((bm, kf), F32)],
    )(*args)


def _pass4(adj_bf, p4_bf, c1, c2, c3, d1, d2, d3, zb, wl4, wlz,
           bl_pad, w54, w5z, bm=400):
    m, k = adj_bf.shape
    nq = w54.shape[1]
    nsteps = m // bm

    def body(a_ref, p_ref, c1_ref, c2_ref, c3_ref, d1_ref, d2_ref, d3_ref,
             zb_ref, wl4_ref, wlz_ref, bl_ref, w54_ref, w5z_ref,
             q_ref, t_ref):
        z4f = jnp.maximum(t_ref[...], 0.0)
        z4_bf = z4f.astype(BF)
        zbv = zb_ref[...]

        t = bl_ref[0:1, :] + c1_ref[...] + c2_ref[...] + c3_ref[...]
        t = t + jnp.dot(z4_bf, wl4_ref[...], preferred_element_type=F32)
        t = t + jnp.dot(zbv, wlz_ref[...], preferred_element_type=F32)
        t = jnp.where(t >= 0, t, 0.01 * t)

        ls = [t[:, c:c + 1] for c in range(5)]
        mx = ls[0]
        for c in range(1, 5):
            mx = jnp.maximum(mx, ls[c])
        es = [jnp.exp(lc - mx) for lc in ls]
        s = es[0] + es[1] + es[2] + es[3] + es[4]
        us = [ec / s for ec in es]
        nrm = jnp.sqrt(us[0] ** 2 + us[1] ** 2 + us[2] ** 2
                       + us[3] ** 2 + us[4] ** 2)
        inv = 1.0 / jnp.maximum(nrm, 1e-12)

        d4 = jnp.dot(z4_bf, w54_ref[...], preferred_element_type=F32)
        dz = jnp.dot(zbv, w5z_ref[...], preferred_element_type=F32)
        acc = (us[0] * inv) * d1_ref[...] + (us[1] * inv) * d2_ref[...] \
            + (us[2] * inv) * d3_ref[...] + (us[3] * inv) * d4 \
            + (us[4] * inv) * dz
        q_ref[...] = acc.astype(BF)

        t_ref[...] = jnp.dot(a_ref[...], p_ref[...],
                             preferred_element_type=F32)

    cur = lambda i: (jnp.minimum(i, nsteps - 1), 0)
    prev = lambda i: (jnp.maximum(i - 1, 0), 0)

    in_specs = [
        pl.BlockSpec((bm, k), cur),
        pl.BlockSpec(p4_bf.shape, lambda i: (0, 0)),
        pl.BlockSpec((bm, 128), prev),
        pl.BlockSpec((bm, 128), prev),
        pl.BlockSpec((bm, 128), prev),
        pl.BlockSpec((bm, nq), prev),
        pl.BlockSpec((bm, nq), prev),
        pl.BlockSpec((bm, nq), prev),
        pl.BlockSpec((bm, zb.shape[1]), prev),
        pl.BlockSpec(wl4.shape, lambda i: (0, 0)),
        pl.BlockSpec(wlz.shape, lambda i: (0, 0)),
        pl.BlockSpec(bl_pad.shape, lambda i: (0, 0)),
        pl.BlockSpec(w54.shape, lambda i: (0, 0)),
        pl.BlockSpec(w5z.shape, lambda i: (0, 0)),
    ]

    return pl.pallas_call(
        body,
        grid=(nsteps + 1,),
        in_specs=in_specs,
        out_specs=pl.BlockSpec((bm, nq), prev),
        out_shape=jax.ShapeDtypeStruct((m, nq), BF),
        scratch_shapes=[pltpu.VMEM((bm, nq), F32)],
    )(adj_bf, p4_bf, c1, c2, c3, d1, d2, d3, zb, wl4, wlz,
      bl_pad, w54, w5z)


def _pass5(adj_bf, q_bf, bm=400):
    m, k = adj_bf.shape
    n = q_bf.shape[1]

    def body(a_ref, q_ref, o_ref):
        acc = jnp.dot(a_ref[...], q_ref[...], preferred_element_type=F32)
        mx = jnp.max(acc, axis=1, keepdims=True)
        e = jnp.exp(acc - mx)
        o_ref[...] = e / jnp.sum(e, axis=1, keepdims=True)

    return pl.pallas_call(
        body,
        grid=(m // bm,),
        in_specs=[
            pl.BlockSpec((bm, k), lambda i: (i, 0)),
            pl.BlockSpec(q_bf.shape, lambda i: (0, 0)),
        ],
        out_specs=pl.BlockSpec((bm, n), lambda i: (i, 0)),
        out_shape=jax.ShapeDtypeStruct((m, n), F32),
    )(adj_bf, q_bf)


def _pad_cols(w, n=128):
    return jnp.pad(w, ((0, 0), (0, n - w.shape[1]))).astype(BF)


def _bias_pad(b):
    return jnp.zeros((8, 128), F32).at[0, : b.shape[0]].set(b)


def kernel(x, adj, h1, h2, h3, z, W1, W2, W3, W4, W5,
           w1, b1, w2, b2, w3, b3, wl, bl):
    ne1 = W1.shape[1]
    ne2 = W2.shape[1]
    ne3 = W3.shape[1]
    nz = W4.shape[1]

    splits = [ne1, ne1 + ne2, ne1 + ne2 + ne3, ne1 + ne2 + ne3 + nz]
    wls = [_pad_cols(p) for p in jnp.split(wl, splits, axis=0)]
    w5s = [p.astype(BF) for p in jnp.split(W5, splits, axis=0)]

    g1, c1, d1, adj_bf = _pass1(
        adj, x.astype(BF), W1.astype(BF), h1,
        _pad_cols(w1[:ne1]), _pad_cols(w1[ne1:]), _bias_pad(b1),
        wls[0], w5s[0])

    g2, c2, d2 = _pass_mid(
        adj_bf, g1, W2.astype(BF), h2,
        _pad_cols(w2[:ne2]), _pad_cols(w2[ne2:]), _bias_pad(b2),
        wls[1], w5s[1])

    p4, c3, d3 = _pass_mid(
        adj_bf, g2, W3.astype(BF), h3,
        _pad_cols(w3[:ne3]), _pad_cols(w3[ne3:]), _bias_pad(b3),
        wls[2], w5s[2], w4_bf=W4.astype(BF))

    q = _pass4(adj_bf, p4, c1, c2, c3, d1, d2, d3, z.astype(BF),
               wls[3], wls[4], _bias_pad(bl), w5s[3], w5s[4])

    return _pass5(adj_bf, q)


# speedup vs baseline: 1.0145x; 1.0121x over previous
import jax
import jax.numpy as jnp
from jax.experimental import pallas as pl
from jax.experimental.pallas import tpu as pltpu

BF = jnp.bfloat16
F32 = jnp.float32


def _gate_coeffs(h_bf, z_bf, wh_ref, wz_ref, b_ref):
    l = (
        jnp.dot(h_bf, wh_ref[...], preferred_element_type=F32)
        + jnp.dot(z_bf, wz_ref[...], preferred_element_type=F32)
        + b_ref[0:1, :]
    )
    l = jnp.where(l >= 0, l, 0.01 * l)
    l0 = l[:, 0:1]
    l1 = l[:, 1:2]
    mx = jnp.maximum(l0, l1)
    e0 = jnp.exp(l0 - mx)
    e1 = jnp.exp(l1 - mx)
    s = e0 + e1
    m0 = e0 / s
    m1 = e1 / s
    inv = 1.0 / jnp.maximum(jnp.sqrt(m0 * m0 + m1 * m1), 1e-12)
    return m0 * inv, m1 * inv


def _pass1(adj_f32, x_bf, w_bf, h_f32, wh, wz, bp, bm=400):
    m, k = adj_f32.shape
    n = w_bf.shape[1]
    kf = x_bf.shape[1]
    nsteps = m // bm

    def body(a_ref, x_ref, w_ref, h_ref, wh_ref, wz_ref, b_ref,
             z_ref, g_ref, abf_ref, t_ref):
        zf = jnp.maximum(
            jnp.dot(t_ref[...].astype(BF), w_ref[...],
                    preferred_element_type=F32), 0.0)
        z_bf = zf.astype(BF)
        z_ref[...] = z_bf
        hf = h_ref[...]
        m0, m1 = _gate_coeffs(hf.astype(BF), z_bf, wh_ref, wz_ref, b_ref)
        g_ref[...] = (m0 * zf + m1 * hf).astype(BF)

        a_bf = a_ref[...].astype(BF)
        abf_ref[...] = a_bf
        t_ref[...] = jnp.dot(a_bf, x_ref[...], preferred_element_type=F32)

    cur = lambda i: (jnp.minimum(i, nsteps - 1), 0)
    prev = lambda i: (jnp.maximum(i - 1, 0), 0)

    return pl.pallas_call(
        body,
        grid=(nsteps + 1,),
        in_specs=[
            pl.BlockSpec((bm, k), cur),
            pl.BlockSpec(x_bf.shape, lambda i: (0, 0)),
            pl.BlockSpec(w_bf.shape, lambda i: (0, 0)),
            pl.BlockSpec((bm, n), prev),
            pl.BlockSpec(wh.shape, lambda i: (0, 0)),
            pl.BlockSpec(wz.shape, lambda i: (0, 0)),
            pl.BlockSpec(bp.shape, lambda i: (0, 0)),
        ],
        out_specs=[
            pl.BlockSpec((bm, n), prev),
            pl.BlockSpec((bm, n), prev),
            pl.BlockSpec((bm, k), cur),
        ],
        out_shape=[
            jax.ShapeDtypeStruct((m, n), BF),
            jax.ShapeDtypeStruct((m, n), BF),
            jax.ShapeDtypeStruct((m, k), BF),
        ],
        scratch_shapes=[pltpu.VMEM((bm, kf), F32)],
    )(adj_f32, x_bf, w_bf, h_f32, wh, wz, bp)


def _pass_mid(adj_bf, f_bf, w_bf, h_f32, wh, wz, bp, w4_bf=None, bm=400):
    m, k = adj_bf.shape
    kf = f_bf.shape[1]
    n = w_bf.shape[1]
    nsteps = m // bm
    emit_p = w4_bf is not None

    def body(a_ref, f_ref, w_ref, h_ref, wh_ref, wz_ref, b_ref, *rest):
        if emit_p:
            w4_ref, z_ref, o_ref, t_ref = rest
        else:
            z_ref, o_ref, t_ref = rest
        zf = jnp.maximum(
            jnp.dot(t_ref[...].astype(BF), w_ref[...],
                    preferred_element_type=F32), 0.0)
        z_bf = zf.astype(BF)
        z_ref[...] = z_bf
        hf = h_ref[...]
        h_bf = hf.astype(BF)
        m0, m1 = _gate_coeffs(h_bf, z_bf, wh_ref, wz_ref, b_ref)
        if emit_p:
            zw = jnp.dot(z_bf, w4_ref[...],
                         preferred_element_type=F32)
            hw = jnp.dot(h_bf, w4_ref[...],
                         preferred_element_type=F32)
            o_ref[...] = (m0 * zw + m1 * hw).astype(BF)
        else:
            o_ref[...] = (m0 * zf + m1 * hf).astype(BF)
        t_ref[...] = jnp.dot(a_ref[...], f_ref[...],
                             preferred_element_type=F32)

    cur = lambda i: (jnp.minimum(i, nsteps - 1), 0)
    prev = lambda i: (jnp.maximum(i - 1, 0), 0)

    in_specs = [
        pl.BlockSpec((bm, k), cur),
        pl.BlockSpec(f_bf.shape, lambda i: (0, 0)),
        pl.BlockSpec(w_bf.shape, lambda i: (0, 0)),
        pl.BlockSpec((bm, n), prev),
        pl.BlockSpec(wh.shape, lambda i: (0, 0)),
        pl.BlockSpec(wz.shape, lambda i: (0, 0)),
        pl.BlockSpec(bp.shape, lambda i: (0, 0)),
    ]
    out_specs = [pl.BlockSpec((bm, n), prev)]
    out_shape = [jax.ShapeDtypeStruct((m, n), BF)]
    args = [adj_bf, f_bf, w_bf, h_f32, wh, wz, bp]
    if emit_p:
        in_specs.append(pl.BlockSpec(w4_bf.shape, lambda i: (0, 0)))
        args.append(w4_bf)
        nq = w4_bf.shape[1]
        out_specs.append(pl.BlockSpec((bm, nq), prev))
        out_shape.append(jax.ShapeDtypeStruct((m, nq), BF))
    else:
        out_specs.append(pl.BlockSpec((bm, n), prev))
        out_shape.append(jax.ShapeDtypeStruct((m, n), BF))

    return pl.pallas_call(
        body,
        grid=(nsteps + 1,),
        in_specs=in_specs,
        out_specs=out_specs,
        out_shape=out_shape,
        scratch_shapes=[pltpu.VMEM((bm, kf), F32)],
    )(*args)


def _pass4(adj_bf, p4_bf, z1, z2, z3, zb, wls, bl_pad, w5s, bm=400):
    m, k = adj_bf.shape
    n_z = [z1.shape[1], z2.shape[1], z3.shape[1], zb.shape[1]]
    nq = w5s[0].shape[1]
    nsteps = m // bm

    def body(a_ref, p_ref, z1_ref, z2_ref, z3_ref, zb_ref,
             wl1_ref, wl2_ref, wl3_ref, wl4_ref, wlz_ref, bl_ref,
             w51_ref, w52_ref, w53_ref, w54_ref, w5z_ref, q_ref, t_ref):
        z4f = jnp.maximum(t_ref[...], 0.0)
        z4_bf = z4f.astype(BF)
        z_blks = (z1_ref[...], z2_ref[...], z3_ref[...], z4_bf, zb_ref[...])
        wl_refs = (wl1_ref, wl2_ref, wl3_ref, wl4_ref, wlz_ref)
        w5_refs = (w51_ref, w52_ref, w53_ref, w54_ref, w5z_ref)

        t = bl_ref[0:1, :]
        for zc, wl_ref in zip(z_blks, wl_refs):
            t = t + jnp.dot(zc, wl_ref[...], preferred_element_type=F32)
        t = jnp.where(t >= 0, t, 0.01 * t)

        ls = [t[:, c:c + 1] for c in range(5)]
        mx = ls[0]
        for c in range(1, 5):
            mx = jnp.maximum(mx, ls[c])
        es = [jnp.exp(lc - mx) for lc in ls]
        s = es[0] + es[1] + es[2] + es[3] + es[4]
        us = [ec / s for ec in es]
        nrm = jnp.sqrt(us[0] ** 2 + us[1] ** 2 + us[2] ** 2
                       + us[3] ** 2 + us[4] ** 2)
        inv = 1.0 / jnp.maximum(nrm, 1e-12)

        acc = jnp.zeros((z4f.shape[0], nq), F32)
        for uc, zc, w5_ref in zip(us, z_blks, w5_refs):
            acc = acc + (uc * inv) * jnp.dot(
                zc, w5_ref[...], preferred_element_type=F32)
        q_ref[...] = acc.astype(BF)

        t_ref[...] = jnp.dot(a_ref[...], p_ref[...],
                             preferred_element_type=F32)

    cur = lambda i: (jnp.minimum(i, nsteps - 1), 0)
    prev = lambda i: (jnp.maximum(i - 1, 0), 0)

    in_specs = [
        pl.BlockSpec((bm, k), cur),
        pl.BlockSpec(p4_bf.shape, lambda i: (0, 0)),
        pl.BlockSpec((bm, n_z[0]), prev),
        pl.BlockSpec((bm, n_z[1]), prev),
        pl.BlockSpec((bm, n_z[2]), prev),
        pl.BlockSpec((bm, n_z[3]), prev),
    ]
    in_specs += [pl.BlockSpec(w.shape, lambda i: (0, 0)) for w in wls]
    in_specs += [pl.BlockSpec(bl_pad.shape, lambda i: (0, 0))]
    in_specs += [pl.BlockSpec(w.shape, lambda i: (0, 0)) for w in w5s]

    return pl.pallas_call(
        body,
        grid=(nsteps + 1,),
        in_specs=in_specs,
        out_specs=pl.BlockSpec((bm, nq), prev),
        out_shape=jax.ShapeDtypeStruct((m, nq), BF),
        scratch_shapes=[pltpu.VMEM((bm, nq), F32)],
    )(adj_bf, p4_bf, z1, z2, z3, zb, *wls, bl_pad, *w5s)


def _pass5(adj_bf, q_bf, bm=400):
    m, k = adj_bf.shape
    n = q_bf.shape[1]

    def body(a_ref, q_ref, o_ref):
        acc = jnp.dot(a_ref[...], q_ref[...], preferred_element_type=F32)
        mx = jnp.max(acc, axis=1, keepdims=True)
        e = jnp.exp(acc - mx)
        o_ref[...] = e / jnp.sum(e, axis=1, keepdims=True)

    return pl.pallas_call(
        body,
        grid=(m // bm,),
        in_specs=[
            pl.BlockSpec((bm, k), lambda i: (i, 0)),
            pl.BlockSpec(q_bf.shape, lambda i: (0, 0)),
        ],
        out_specs=pl.BlockSpec((bm, n), lambda i: (i, 0)),
        out_shape=jax.ShapeDtypeStruct((m, n), F32),
    )(adj_bf, q_bf)


def _pad_cols(w, n=128):
    return jnp.pad(w, ((0, 0), (0, n - w.shape[1]))).astype(BF)


def _bias_pad(b):
    return jnp.zeros((8, 128), F32).at[0, : b.shape[0]].set(b)


def kernel(x, adj, h1, h2, h3, z, W1, W2, W3, W4, W5,
           w1, b1, w2, b2, w3, b3, wl, bl):
    ne1 = W1.shape[1]
    ne2 = W2.shape[1]
    ne3 = W3.shape[1]
    nz = W4.shape[1]

    z1, g1, adj_bf = _pass1(
        adj, x.astype(BF), W1.astype(BF), h1,
        _pad_cols(w1[:ne1]), _pad_cols(w1[ne1:]), _bias_pad(b1))

    z2, g2 = _pass_mid(
        adj_bf, g1, W2.astype(BF), h2,
        _pad_cols(w2[:ne2]), _pad_cols(w2[ne2:]), _bias_pad(b2))

    z3, p4 = _pass_mid(
        adj_bf, g2, W3.astype(BF), h3,
        _pad_cols(w3[:ne3]), _pad_cols(w3[ne3:]), _bias_pad(b3),
        w4_bf=W4.astype(BF))

    splits = [ne1, ne1 + ne2, ne1 + ne2 + ne3, ne1 + ne2 + ne3 + nz]
    wls = [_pad_cols(p).astype(BF) for p in jnp.split(wl, splits, axis=0)]
    w5s = [p.astype(BF) for p in jnp.split(W5, splits, axis=0)]
    q = _pass4(adj_bf, p4, z1, z2, z3, z.astype(BF), wls, _bias_pad(bl), w5s)

    return _pass5(adj_bf, q)
